# baseline (speedup 1.0000x reference)
import math

import jax
import jax.numpy as jnp
from jax import lax
from jax.experimental import pallas as pl
from jax.experimental.pallas import tpu as pltpu
from jax.experimental.pallas import tpu_sc as plsc

D_EMB = 64
NUM_WORKERS = 32
GROUP = 128
NB = 4
SCALE = math.sqrt(D_EMB)


def _make_sc_kernel(n_idx):
    per_w = n_idx // NUM_WORKERS
    n_groups = per_w // GROUP
    n_outer = n_groups // NB

    mesh = plsc.VectorSubcoreMesh(core_axis_name="c", subcore_axis_name="s")

    def body(x_hbm, lut_hbm, out_hbm, idx_v, in_bufs, out_bufs, sem_in, sem_out):
        nc = lax.axis_size("c")
        wid = lax.axis_index("s") * nc + lax.axis_index("c")
        base = wid * per_w

        pltpu.sync_copy(x_hbm.at[pl.ds(base, per_w)], idx_v)

        def gather_desc(g, b):
            return pltpu.make_async_copy(
                lut_hbm.at[idx_v.at[pl.ds(g * GROUP, GROUP)]],
                in_bufs[b],
                sem_in[b],
            )

        def out_desc(g, b):
            return pltpu.make_async_copy(
                out_bufs[b],
                out_hbm.at[pl.ds(base + g * GROUP, GROUP)],
                sem_out[b],
            )

        for b in range(NB):
            gather_desc(b, b).start()

        def scale_group(b):
            in_ref = in_bufs[b]
            out_ref = out_bufs[b]

            def row_body(r, carry):
                for rr in range(4):
                    row = r * 4 + rr
                    for j in range(D_EMB // 16):
                        out_ref[row, pl.ds(j * 16, 16)] = (
                            in_ref[row, pl.ds(j * 16, 16)] * SCALE
                        )
                return carry

            lax.fori_loop(0, GROUP // 4, row_body, None)

        def outer_body(o, carry):
            for b in range(NB):
                g = o * NB + b
                gather_desc(g, b).wait()

                @pl.when(o > 0)
                def _wait_prev_out():
                    out_desc(g, b).wait()

                scale_group(b)
                out_desc(g, b).start()

                @pl.when(o + 1 < n_outer)
                def _next_gather():
                    gather_desc(g + NB, b).start()
            return carry

        lax.fori_loop(0, n_outer, outer_body, None)

        for b in range(NB):
            out_desc((n_outer - 1) * NB + b, b).wait()

    grid_kernel = pl.kernel(
        body,
        out_type=jax.ShapeDtypeStruct((n_idx, D_EMB), jnp.float32),
        mesh=mesh,
        scratch_types=[
            pltpu.VMEM((per_w,), jnp.int32),
            [pltpu.VMEM((GROUP, D_EMB), jnp.float32) for _ in range(NB)],
            [pltpu.VMEM((GROUP, D_EMB), jnp.float32) for _ in range(NB)],
            [pltpu.SemaphoreType.DMA for _ in range(NB)],
            [pltpu.SemaphoreType.DMA for _ in range(NB)],
        ],
        compiler_params=pltpu.CompilerParams(use_tc_tiling_on_sc=False),
    )
    return grid_kernel


@jax.jit
def kernel(x, lut):
    b, l = x.shape
    flat_idx = x.reshape(b * l).astype(jnp.int32)
    out = _make_sc_kernel(b * l)(flat_idx, lut)
    return out.reshape(b, l, D_EMB)
